# Initial kernel scaffold; baseline (speedup 1.0000x reference)
import jax
import jax.numpy as jnp
from jax.experimental import pallas as pl


def kernel(edge_index, edge_weight, user_feat, item_feat, W, b):
    raise NotImplementedError("write your pallas kernel here")


# speedup vs baseline: 2.8246x; 2.8246x over previous
import functools

import jax
import jax.numpy as jnp
from jax import lax
from jax.experimental import pallas as pl
from jax.experimental.pallas import tpu as pltpu
from jax.experimental.pallas import tpu_sc as plsc

L = 16
NC = 2
NS = 16
CHUNK = 80


def _linear_kernel(x_ref, wt_ref, b_ref, o_ref):
    o_ref[...] = (
        jnp.dot(x_ref[...], wt_ref[...], preferred_element_type=jnp.float32)
        + b_ref[...]
    )


def _combine_kernel(p0_ref, p1_ref, o_ref):
    o_ref[...] = p0_ref[...] + p1_ref[...]


def _make_scatter(n_nodes, d, n_edges):
    n_workers = NC * NS
    e_per_w = n_edges // n_workers
    n_chunks = e_per_w // CHUNK
    zrows = CHUNK
    stride = ((n_nodes + NS - 1) // NS + zrows - 1) // zrows * zrows
    last_rows = n_nodes - (NS - 1) * stride
    assert e_per_w * n_workers == n_edges
    assert n_chunks * CHUNK == e_per_w
    assert 0 < last_rows <= stride
    assert stride % zrows == 0 and last_rows % zrows == 0

    mesh = plsc.VectorSubcoreMesh(core_axis_name="c", subcore_axis_name="s",
                                  num_cores=NC, num_subcores=NS)

    @functools.partial(
        pl.kernel,
        out_type=jax.ShapeDtypeStruct((2 * n_nodes, d), jnp.float32),
        mesh=mesh,
        scratch_types=[
            pltpu.VMEM((CHUNK,), jnp.int32),
            pltpu.VMEM((CHUNK,), jnp.int32),
            pltpu.VMEM((CHUNK, L), jnp.float32),
            pltpu.VMEM((CHUNK, d), jnp.float32),
            pltpu.VMEM((zrows, d), jnp.float32),
            pltpu.VMEM_SHARED((NS * stride, d), jnp.float32),
            pltpu.SemaphoreType.DMA,
        ],
    )
    def scatter(embs_hbm, src_hbm, dst_hbm, w_hbm, feat_hbm, out_hbm,
                srcv, dstv, wv, rows, zbuf, acc, sem):
        c = lax.axis_index("c")
        s = lax.axis_index("s")

        r0 = s * stride

        @pl.when(c == 0)
        def _():
            @pl.when(s < NS - 1)
            def _():
                pltpu.sync_copy(feat_hbm.at[pl.ds(r0, stride)],
                                acc.at[pl.ds(r0, stride)])

            @pl.when(s == NS - 1)
            def _():
                pltpu.sync_copy(feat_hbm.at[pl.ds(r0, last_rows)],
                                acc.at[pl.ds(r0, last_rows)])

        @pl.when(c != 0)
        def _():
            zero = jnp.zeros((L,), jnp.float32)

            def zrow(i, carry):
                for j in range(d // L):
                    zbuf[i, pl.ds(j * L, L)] = zero
                return carry

            lax.fori_loop(0, zrows, zrow, 0)

            @pl.when(s < NS - 1)
            def _():
                for t in range(stride // zrows):
                    pltpu.sync_copy(zbuf, acc.at[pl.ds(r0 + t * zrows, zrows)])

            @pl.when(s == NS - 1)
            def _():
                for t in range(last_rows // zrows):
                    pltpu.sync_copy(zbuf, acc.at[pl.ds(r0 + t * zrows, zrows)])

        plsc.subcore_barrier()

        base = (c * NS + s) * e_per_w

        def chunk_body(g, carry):
            e0 = base + g * CHUNK
            pltpu.sync_copy(src_hbm.at[pl.ds(e0, CHUNK)], srcv)
            pltpu.sync_copy(dst_hbm.at[pl.ds(e0, CHUNK)], dstv)
            pltpu.sync_copy(w_hbm.at[pl.ds(e0, CHUNK)], wv)
            pltpu.async_copy(embs_hbm.at[srcv], rows, sem).wait()

            def row_body(i, rcarry):
                wb = wv[i, :]
                for j in range(d // L):
                    sl = pl.ds(j * L, L)
                    rows[i, sl] = rows[i, sl] * wb
                return rcarry

            lax.fori_loop(0, CHUNK, row_body, 0)
            pltpu.sync_copy(rows, acc.at[dstv], add=True)
            return carry

        lax.fori_loop(0, n_chunks, chunk_body, 0)

        plsc.subcore_barrier()

        @pl.when(s < NS - 1)
        def _():
            pltpu.sync_copy(acc.at[pl.ds(r0, stride)],
                            out_hbm.at[pl.ds(c * n_nodes + r0, stride)])

        @pl.when(s == NS - 1)
        def _():
            pltpu.sync_copy(acc.at[pl.ds(r0, last_rows)],
                            out_hbm.at[pl.ds(c * n_nodes + r0, last_rows)])

    return scatter


def kernel(edge_index, edge_weight, user_feat, item_feat, W, b):
    n_users, d = user_feat.shape
    n_items = item_feat.shape[0]
    n_nodes = n_users + n_items
    n_edges = edge_weight.shape[0]

    feat_all = jnp.concatenate([user_feat, item_feat], axis=0)
    dst = edge_index[0]
    src = edge_index[1]

    blk = 1000
    embs = pl.pallas_call(
        _linear_kernel,
        grid=(n_nodes // blk,),
        in_specs=[
            pl.BlockSpec((blk, d), lambda i: (i, 0)),
            pl.BlockSpec((d, d), lambda i: (0, 0)),
            pl.BlockSpec((1, d), lambda i: (0, 0)),
        ],
        out_specs=pl.BlockSpec((blk, d), lambda i: (i, 0)),
        out_shape=jax.ShapeDtypeStruct((n_nodes, d), jnp.float32),
    )(feat_all, W.T, b.reshape(1, d))

    w_exp = jnp.broadcast_to(edge_weight[:, None], (n_edges, L))
    partials = _make_scatter(n_nodes, d, n_edges)(
        embs, src, dst, w_exp, feat_all)

    out_all = pl.pallas_call(
        _combine_kernel,
        grid=(n_nodes // blk,),
        in_specs=[
            pl.BlockSpec((blk, d), lambda i: (i, 0)),
            pl.BlockSpec((blk, d), lambda i: (i, 0)),
        ],
        out_specs=pl.BlockSpec((blk, d), lambda i: (i, 0)),
        out_shape=jax.ShapeDtypeStruct((n_nodes, d), jnp.float32),
    )(partials[:n_nodes], partials[n_nodes:])

    return (out_all[:n_users], out_all[n_users:])


# speedup vs baseline: 3.8126x; 1.3498x over previous
import functools

import jax
import jax.numpy as jnp
from jax import lax
from jax.experimental import pallas as pl
from jax.experimental.pallas import tpu as pltpu
from jax.experimental.pallas import tpu_sc as plsc

L = 16
NC = 2
NS = 16
CHUNK = 80


def _linear_kernel(x_ref, wt_ref, b_ref, o_ref):
    o_ref[...] = (
        jnp.dot(x_ref[...], wt_ref[...], preferred_element_type=jnp.float32)
        + b_ref[...]
    )


def _combine_kernel(p0u_ref, p1u_ref, p0i_ref, p1i_ref, ou_ref, oi_ref):
    ou_ref[...] = p0u_ref[...] + p1u_ref[...]
    oi_ref[...] = p0i_ref[...] + p1i_ref[...]


def _make_scatter(n_nodes, d, n_edges):
    n_workers = NC * NS
    e_per_w = n_edges // n_workers
    n_chunks = e_per_w // CHUNK
    zrows = CHUNK
    stride = ((n_nodes + NS - 1) // NS + zrows - 1) // zrows * zrows
    last_rows = n_nodes - (NS - 1) * stride
    assert e_per_w * n_workers == n_edges
    assert n_chunks * CHUNK == e_per_w and n_chunks >= 3
    assert 0 < last_rows <= stride
    assert stride % zrows == 0 and last_rows % zrows == 0

    mesh = plsc.VectorSubcoreMesh(core_axis_name="c", subcore_axis_name="s",
                                  num_cores=NC, num_subcores=NS)

    @functools.partial(
        pl.kernel,
        out_type=jax.ShapeDtypeStruct((2 * n_nodes, d), jnp.float32),
        mesh=mesh,
        scratch_types=[
            pltpu.VMEM((2, CHUNK), jnp.int32),
            pltpu.VMEM((CHUNK,), jnp.int32),
            pltpu.VMEM((CHUNK,), jnp.int32),
            pltpu.VMEM((CHUNK,), jnp.int32),
            pltpu.VMEM((CHUNK,), jnp.int32),
            pltpu.VMEM((2, CHUNK), jnp.float32),
            pltpu.VMEM((2, CHUNK, d), jnp.float32),
            pltpu.VMEM((2, CHUNK, d), jnp.float32),
            pltpu.VMEM_SHARED((NS * stride, d), jnp.float32),
            pltpu.SemaphoreType.DMA,
            pltpu.SemaphoreType.DMA,
            pltpu.SemaphoreType.DMA,
            pltpu.SemaphoreType.DMA,
            pltpu.SemaphoreType.DMA,
            pltpu.SemaphoreType.DMA,
        ],
    )
    def scatter(embs_hbm, src_hbm, dst_hbm, w_hbm, feat_hbm, out_hbm,
                srcv, dstva, dstvb, dstvc, dstvd, wring, gbuf, sbuf, acc,
                isem0, isem1, gsem0, gsem1, ssem0, ssem1):
        c = lax.axis_index("c")
        s = lax.axis_index("s")

        r0 = s * stride

        @pl.when(c == 0)
        def _():
            @pl.when(s < NS - 1)
            def _():
                pltpu.sync_copy(feat_hbm.at[pl.ds(r0, stride)],
                                acc.at[pl.ds(r0, stride)])

            @pl.when(s == NS - 1)
            def _():
                pltpu.sync_copy(feat_hbm.at[pl.ds(r0, last_rows)],
                                acc.at[pl.ds(r0, last_rows)])

        @pl.when(c != 0)
        def _():
            zero = jnp.zeros((L,), jnp.float32)

            def zrow(i, carry):
                for j in range(d // L):
                    sbuf[0, i, pl.ds(j * L, L)] = zero
                return carry

            lax.fori_loop(0, zrows, zrow, 0)

            @pl.when(s < NS - 1)
            def _():
                for t in range(stride // zrows):
                    pltpu.sync_copy(sbuf.at[0],
                                    acc.at[pl.ds(r0 + t * zrows, zrows)])

            @pl.when(s == NS - 1)
            def _():
                for t in range(last_rows // zrows):
                    pltpu.sync_copy(sbuf.at[0],
                                    acc.at[pl.ds(r0 + t * zrows, zrows)])

        plsc.subcore_barrier()

        base = (c * NS + s) * e_per_w

        def wait_rows(dst_ref, sem):
            pltpu.make_async_copy(embs_hbm.at[pl.ds(0, CHUNK)],
                                  dst_ref, sem).wait()

        def wait_idx(dst_ref, sem):
            pltpu.make_async_copy(src_hbm.at[pl.ds(0, CHUNK)],
                                  dst_ref, sem).wait()

        pltpu.async_copy(src_hbm.at[pl.ds(base, CHUNK)], srcv.at[0], isem0)
        pltpu.async_copy(dst_hbm.at[pl.ds(base, CHUNK)], dstva, isem0)
        pltpu.async_copy(w_hbm.at[pl.ds(base, CHUNK)], wring.at[0], isem0)
        pltpu.async_copy(src_hbm.at[pl.ds(base + CHUNK, CHUNK)],
                         srcv.at[1], isem1)
        pltpu.async_copy(dst_hbm.at[pl.ds(base + CHUNK, CHUNK)],
                         dstvb, isem1)
        pltpu.async_copy(w_hbm.at[pl.ds(base + CHUNK, CHUNK)],
                         wring.at[1], isem1)
        for _ in range(3):
            wait_idx(srcv.at[0], isem0)
        pltpu.async_copy(embs_hbm.at[srcv.at[0]], gbuf.at[0], gsem0)

        def body(g, carry):
            r = lax.rem(g, 2)
            q = lax.rem(g, 4)

            @pl.when(r == 0)
            def _():
                wait_rows(gbuf.at[0], gsem0)

            @pl.when(r == 1)
            def _():
                wait_rows(gbuf.at[1], gsem1)

            @pl.when((g >= 2) & (r == 0))
            def _():
                wait_rows(sbuf.at[0], ssem0)

            @pl.when((g >= 2) & (r == 1))
            def _():
                wait_rows(sbuf.at[1], ssem1)

            @pl.when((g + 1 < n_chunks) & (r == 0))
            def _():
                for _ in range(3):
                    wait_idx(srcv.at[1], isem1)
                pltpu.async_copy(embs_hbm.at[srcv.at[1]], gbuf.at[1], gsem1)

            @pl.when((g + 1 < n_chunks) & (r == 1))
            def _():
                for _ in range(3):
                    wait_idx(srcv.at[0], isem0)
                pltpu.async_copy(embs_hbm.at[srcv.at[0]], gbuf.at[0], gsem0)

            gdn = lax.GatherDimensionNumbers(
                offset_dims=(), collapsed_slice_dims=(0,),
                start_index_map=(0,))

            def kgroup(k, kcarry):
                row0 = k * L
                w16 = wring[r, pl.ds(row0, L)]
                for rr in range(L):
                    wb = lax.gather(
                        w16, jnp.full((L, 1), rr, jnp.int32), gdn,
                        slice_sizes=(1,),
                        mode=lax.GatherScatterMode.PROMISE_IN_BOUNDS)
                    for j in range(d // L):
                        sl = pl.ds(j * L, L)
                        sbuf[r, row0 + rr, sl] = gbuf[r, row0 + rr, sl] * wb
                return kcarry

            lax.fori_loop(0, CHUNK // L, kgroup, 0)

            @pl.when(q == 0)
            def _():
                pltpu.async_copy(sbuf.at[0], acc.at[dstva], ssem0, add=True)

            @pl.when(q == 1)
            def _():
                pltpu.async_copy(sbuf.at[1], acc.at[dstvb], ssem1, add=True)

            @pl.when(q == 2)
            def _():
                pltpu.async_copy(sbuf.at[0], acc.at[dstvc], ssem0, add=True)

            @pl.when(q == 3)
            def _():
                pltpu.async_copy(sbuf.at[1], acc.at[dstvd], ssem1, add=True)

            e2 = base + (g + 2) * CHUNK
            live = g + 2 < n_chunks

            @pl.when(live & (q == 0))
            def _():
                pltpu.async_copy(src_hbm.at[pl.ds(e2, CHUNK)],
                                 srcv.at[0], isem0)
                pltpu.async_copy(dst_hbm.at[pl.ds(e2, CHUNK)], dstvc, isem0)
                pltpu.async_copy(w_hbm.at[pl.ds(e2, CHUNK)],
                                 wring.at[0], isem0)

            @pl.when(live & (q == 1))
            def _():
                pltpu.async_copy(src_hbm.at[pl.ds(e2, CHUNK)],
                                 srcv.at[1], isem1)
                pltpu.async_copy(dst_hbm.at[pl.ds(e2, CHUNK)], dstvd, isem1)
                pltpu.async_copy(w_hbm.at[pl.ds(e2, CHUNK)],
                                 wring.at[1], isem1)

            @pl.when(live & (q == 2))
            def _():
                pltpu.async_copy(src_hbm.at[pl.ds(e2, CHUNK)],
                                 srcv.at[0], isem0)
                pltpu.async_copy(dst_hbm.at[pl.ds(e2, CHUNK)], dstva, isem0)
                pltpu.async_copy(w_hbm.at[pl.ds(e2, CHUNK)],
                                 wring.at[0], isem0)

            @pl.when(live & (q == 3))
            def _():
                pltpu.async_copy(src_hbm.at[pl.ds(e2, CHUNK)],
                                 srcv.at[1], isem1)
                pltpu.async_copy(dst_hbm.at[pl.ds(e2, CHUNK)], dstvb, isem1)
                pltpu.async_copy(w_hbm.at[pl.ds(e2, CHUNK)],
                                 wring.at[1], isem1)

            return carry

        lax.fori_loop(0, n_chunks, body, 0)

        wait_rows(sbuf.at[0], ssem0)
        wait_rows(sbuf.at[1], ssem1)

        plsc.subcore_barrier()

        @pl.when(s < NS - 1)
        def _():
            pltpu.sync_copy(acc.at[pl.ds(r0, stride)],
                            out_hbm.at[pl.ds(c * n_nodes + r0, stride)])

        @pl.when(s == NS - 1)
        def _():
            pltpu.sync_copy(acc.at[pl.ds(r0, last_rows)],
                            out_hbm.at[pl.ds(c * n_nodes + r0, last_rows)])

    return scatter


def kernel(edge_index, edge_weight, user_feat, item_feat, W, b):
    n_users, d = user_feat.shape
    n_items = item_feat.shape[0]
    n_nodes = n_users + n_items
    n_edges = edge_weight.shape[0]

    feat_all = jnp.concatenate([user_feat, item_feat], axis=0)
    dst = edge_index[0]
    src = edge_index[1]

    blk = 1000
    embs = pl.pallas_call(
        _linear_kernel,
        grid=(n_nodes // blk,),
        in_specs=[
            pl.BlockSpec((blk, d), lambda i: (i, 0)),
            pl.BlockSpec((d, d), lambda i: (0, 0)),
            pl.BlockSpec((1, d), lambda i: (0, 0)),
        ],
        out_specs=pl.BlockSpec((blk, d), lambda i: (i, 0)),
        out_shape=jax.ShapeDtypeStruct((n_nodes, d), jnp.float32),
    )(feat_all, W.T, b.reshape(1, d))

    partials = _make_scatter(n_nodes, d, n_edges)(
        embs, src, dst, edge_weight, feat_all)

    cblk = 1000
    gu = n_users // cblk
    gn = n_nodes // cblk
    out_user, out_item = pl.pallas_call(
        _combine_kernel,
        grid=(gu,),
        in_specs=[
            pl.BlockSpec((cblk, d), lambda i: (i, 0)),
            pl.BlockSpec((cblk, d), lambda i: (i + gn, 0)),
            pl.BlockSpec((cblk, d), lambda i: (i + gu, 0)),
            pl.BlockSpec((cblk, d), lambda i: (i + gn + gu, 0)),
        ],
        out_specs=[
            pl.BlockSpec((cblk, d), lambda i: (i, 0)),
            pl.BlockSpec((cblk, d), lambda i: (i, 0)),
        ],
        out_shape=[
            jax.ShapeDtypeStruct((n_users, d), jnp.float32),
            jax.ShapeDtypeStruct((n_items, d), jnp.float32),
        ],
    )(partials, partials, partials, partials)

    return (out_user, out_item)


# speedup vs baseline: 11.0752x; 2.9049x over previous
import functools

import jax
import jax.numpy as jnp
from jax import lax
from jax.experimental import pallas as pl
from jax.experimental.pallas import tpu as pltpu
from jax.experimental.pallas import tpu_sc as plsc

L = 16
NC = 2
NS = 16
CHUNK = 80


def _linear_kernel(x_ref, wt_ref, b_ref, o_ref):
    o_ref[...] = (
        jnp.dot(x_ref[...], wt_ref[...], preferred_element_type=jnp.float32)
        + b_ref[...]
    )


def _combine_kernel(p0u_ref, p1u_ref, p0i_ref, p1i_ref, ou_ref, oi_ref):
    ou_ref[...] = p0u_ref[...] + p1u_ref[...]
    oi_ref[...] = p0i_ref[...] + p1i_ref[...]


def _make_scatter(n_nodes, d, n_edges):
    n_workers = NC * NS
    e_per_w = n_edges // n_workers
    n_chunks = e_per_w // CHUNK
    zrows = CHUNK
    stride = ((n_nodes + NS - 1) // NS + zrows - 1) // zrows * zrows
    last_rows = n_nodes - (NS - 1) * zrows * (stride // zrows)
    last_rows = n_nodes - (NS - 1) * stride
    assert e_per_w * n_workers == n_edges
    assert n_chunks * CHUNK == e_per_w and n_chunks >= 6
    assert 0 < last_rows <= stride
    assert stride % zrows == 0 and last_rows % zrows == 0

    mesh = plsc.VectorSubcoreMesh(core_axis_name="c", subcore_axis_name="s",
                                  num_cores=NC, num_subcores=NS)

    @functools.partial(
        pl.kernel,
        out_type=jax.ShapeDtypeStruct((2 * n_nodes, d), jnp.float32),
        mesh=mesh,
        scratch_types=(
            [pltpu.VMEM((CHUNK,), jnp.int32) for _ in range(4)]
            + [pltpu.VMEM((CHUNK,), jnp.int32) for _ in range(4)]
            + [pltpu.VMEM((CHUNK,), jnp.float32) for _ in range(4)]
            + [pltpu.VMEM((CHUNK, d), jnp.float32) for _ in range(4)]
            + [pltpu.VMEM_SHARED((NS * stride, d), jnp.float32)]
            + [pltpu.SemaphoreType.DMA for _ in range(12)]
        ),
    )
    def scatter(embs_hbm, src_hbm, dst_hbm, w_hbm, feat_hbm, out_hbm,
                *refs):
        srcvs = refs[0:4]
        dstvs = refs[4:8]
        wrs = refs[8:12]
        bufs = refs[12:16]
        acc = refs[16]
        isems = refs[17:21]
        gsems = refs[21:25]
        ssems = refs[25:29]

        c = lax.axis_index("c")
        s = lax.axis_index("s")

        r0 = s * stride

        @pl.when(c == 0)
        def _():
            @pl.when(s < NS - 1)
            def _():
                pltpu.sync_copy(feat_hbm.at[pl.ds(r0, stride)],
                                acc.at[pl.ds(r0, stride)])

            @pl.when(s == NS - 1)
            def _():
                pltpu.sync_copy(feat_hbm.at[pl.ds(r0, last_rows)],
                                acc.at[pl.ds(r0, last_rows)])

        @pl.when(c != 0)
        def _():
            zero = jnp.zeros((L,), jnp.float32)
            zb = bufs[0]

            def zrow(i, carry):
                for j in range(d // L):
                    zb[i, pl.ds(j * L, L)] = zero
                return carry

            lax.fori_loop(0, zrows, zrow, 0)

            @pl.when(s < NS - 1)
            def _():
                for t in range(stride // zrows):
                    pltpu.sync_copy(zb, acc.at[pl.ds(r0 + t * zrows, zrows)])

            @pl.when(s == NS - 1)
            def _():
                for t in range(last_rows // zrows):
                    pltpu.sync_copy(zb, acc.at[pl.ds(r0 + t * zrows, zrows)])

        plsc.subcore_barrier()

        base = (c * NS + s) * e_per_w

        def wait_rows(dst_ref, sem):
            pltpu.make_async_copy(embs_hbm.at[pl.ds(0, CHUNK)],
                                  dst_ref, sem).wait()

        def wait_idx(dst_ref, sem):
            pltpu.make_async_copy(src_hbm.at[pl.ds(0, CHUNK)],
                                  dst_ref, sem).wait()

        gdn = lax.GatherDimensionNumbers(
            offset_dims=(), collapsed_slice_dims=(0,), start_index_map=(0,))

        def do_scale(bf, wv):
            def kgroup(k, kcarry):
                row0 = k * L
                w16 = wv[pl.ds(row0, L)]
                for rr in range(L):
                    wb = lax.gather(
                        w16, jnp.full((L, 1), rr, jnp.int32), gdn,
                        slice_sizes=(1,),
                        mode=lax.GatherScatterMode.PROMISE_IN_BOUNDS)
                    for j in range(d // L):
                        sl = pl.ds(j * L, L)
                        bf[row0 + rr, sl] = bf[row0 + rr, sl] * wb
                return kcarry

            lax.fori_loop(0, CHUNK // L, kgroup, 0)

        for x in (0, 1):
            e = base + x * CHUNK
            pltpu.async_copy(src_hbm.at[pl.ds(e, CHUNK)], srcvs[x], isems[x])
            pltpu.async_copy(dst_hbm.at[pl.ds(e, CHUNK)], dstvs[x], isems[x])
            pltpu.async_copy(w_hbm.at[pl.ds(e, CHUNK)], wrs[x], isems[x])
        for x in (2, 3):
            e = base + x * CHUNK
            pltpu.async_copy(src_hbm.at[pl.ds(e, CHUNK)], srcvs[x], isems[x])
            pltpu.async_copy(w_hbm.at[pl.ds(e, CHUNK)], wrs[x], isems[x])
        for x in (0, 1):
            for _ in range(3):
                wait_idx(srcvs[x], isems[x])
            pltpu.async_copy(embs_hbm.at[srcvs[x]], bufs[x], gsems[x])

        def body(g, carry):
            u = lax.rem(g, 4)

            for si in range(4):
                s2 = (si + 2) % 4

                @pl.when((g >= 2) & (u == si))
                def _(bf=bufs[s2], sem=ssems[s2]):
                    wait_rows(bf, sem)

                @pl.when((g + 2 < n_chunks) & (u == si))
                def _(sv=srcvs[s2], bf=bufs[s2], isem=isems[s2],
                      gsem=gsems[s2]):
                    wait_idx(sv, isem)
                    wait_idx(sv, isem)
                    pltpu.async_copy(embs_hbm.at[sv], bf, gsem)

            for si in range(4):
                @pl.when(u == si)
                def _(bf=bufs[si], gsem=gsems[si], wv=wrs[si]):
                    wait_rows(bf, gsem)
                    do_scale(bf, wv)

            for si in range(4):
                @pl.when(u == si)
                def _(sv=srcvs[si], dv=dstvs[si], bf=bufs[si],
                      isem=isems[si], ssem=ssems[si]):
                    @pl.when(g >= 2)
                    def _():
                        wait_idx(sv, isem)
                    pltpu.async_copy(bf, acc.at[dv], ssem, add=True)

            e4 = base + (g + 4) * CHUNK
            e2 = base + (g + 2) * CHUNK
            for si in range(4):
                s2 = (si + 2) % 4

                @pl.when((g + 4 < n_chunks) & (u == si))
                def _(sv=srcvs[si], wv=wrs[si], isem=isems[si]):
                    pltpu.async_copy(src_hbm.at[pl.ds(e4, CHUNK)], sv, isem)
                    pltpu.async_copy(w_hbm.at[pl.ds(e4, CHUNK)], wv, isem)

                @pl.when((g + 2 < n_chunks) & (u == si))
                def _(dv=dstvs[s2], isem=isems[s2]):
                    pltpu.async_copy(dst_hbm.at[pl.ds(e2, CHUNK)], dv, isem)

            return carry

        lax.fori_loop(0, n_chunks, body, 0)

        for x in (n_chunks - 2, n_chunks - 1):
            wait_rows(bufs[x % 4], ssems[x % 4])

        plsc.subcore_barrier()

        @pl.when(s < NS - 1)
        def _():
            pltpu.sync_copy(acc.at[pl.ds(r0, stride)],
                            out_hbm.at[pl.ds(c * n_nodes + r0, stride)])

        @pl.when(s == NS - 1)
        def _():
            pltpu.sync_copy(acc.at[pl.ds(r0, last_rows)],
                            out_hbm.at[pl.ds(c * n_nodes + r0, last_rows)])

    return scatter


def kernel(edge_index, edge_weight, user_feat, item_feat, W, b):
    n_users, d = user_feat.shape
    n_items = item_feat.shape[0]
    n_nodes = n_users + n_items
    n_edges = edge_weight.shape[0]

    feat_all = jnp.concatenate([user_feat, item_feat], axis=0)
    dst = edge_index[0]
    src = edge_index[1]

    blk = 1000
    embs = pl.pallas_call(
        _linear_kernel,
        grid=(n_nodes // blk,),
        in_specs=[
            pl.BlockSpec((blk, d), lambda i: (i, 0)),
            pl.BlockSpec((d, d), lambda i: (0, 0)),
            pl.BlockSpec((1, d), lambda i: (0, 0)),
        ],
        out_specs=pl.BlockSpec((blk, d), lambda i: (i, 0)),
        out_shape=jax.ShapeDtypeStruct((n_nodes, d), jnp.float32),
    )(feat_all, W.T, b.reshape(1, d))

    partials = _make_scatter(n_nodes, d, n_edges)(
        embs, src, dst, edge_weight, feat_all)

    cblk = 1000
    gu = n_users // cblk
    gn = n_nodes // cblk
    out_user, out_item = pl.pallas_call(
        _combine_kernel,
        grid=(gu,),
        in_specs=[
            pl.BlockSpec((cblk, d), lambda i: (i, 0)),
            pl.BlockSpec((cblk, d), lambda i: (i + gn, 0)),
            pl.BlockSpec((cblk, d), lambda i: (i + gu, 0)),
            pl.BlockSpec((cblk, d), lambda i: (i + gn + gu, 0)),
        ],
        out_specs=[
            pl.BlockSpec((cblk, d), lambda i: (i, 0)),
            pl.BlockSpec((cblk, d), lambda i: (i, 0)),
        ],
        out_shape=[
            jax.ShapeDtypeStruct((n_users, d), jnp.float32),
            jax.ShapeDtypeStruct((n_items, d), jnp.float32),
        ],
    )(partials, partials, partials, partials)

    return (out_user, out_item)
